# baseline (speedup 1.0000x reference)
import functools

import jax
import jax.numpy as jnp
from jax import lax
from jax.experimental import pallas as pl
from jax.experimental.pallas import tpu as pltpu
from jax.experimental.pallas import tpu_sc as plsc

_K = 64
_N = 32768
_D = 1024
_NC = 2
_NS = 16
_NW = _NC * _NS
_RPW = _N // _NW
_NSPLIT = 4
_CS = _RPW // _NSPLIT

_mesh = plsc.VectorSubcoreMesh(core_axis_name="c", subcore_axis_name="s")


@functools.partial(
    pl.kernel,
    out_type=jax.ShapeDtypeStruct((_N, _D), jnp.float32),
    mesh=_mesh,
    scratch_types=[
        pltpu.VMEM((_RPW,), jnp.int32),
        pltpu.VMEM((16, _D), jnp.float32),
        pltpu.SemaphoreType.DMA,
        pltpu.SemaphoreType.DMA,
    ],
)
def _sc_select(x_hbm, y_hbm, out_hbm, lab_v, zrows_v, csem, zsem):
    wid = lax.axis_index("s") * _NC + lax.axis_index("c")
    base = wid * _RPW

    for p in range(_NSPLIT):
        pltpu.async_copy(
            y_hbm.at[pl.ds(base + p * _CS, _CS)],
            out_hbm.at[pl.ds(base + p * _CS, _CS)],
            csem,
        )

    pltpu.sync_copy(x_hbm.at[pl.ds(base, _RPW)], lab_v)

    def scan16(i, acc):
        lv = lab_v[pl.ds(i * 16, 16)]
        ok = jnp.where((lv >= 0) & (lv < _K), 1, 0)
        return acc & ok

    all_ok16 = lax.fori_loop(
        0, _RPW // 16, scan16, jnp.ones((16,), jnp.int32)
    )
    ok_s = all_ok16[0]
    for l in range(1, 16):
        ok_s = ok_s & all_ok16[l]
    all_ok = ok_s == 1

    for p in range(_NSPLIT):
        pltpu.make_async_copy(
            y_hbm.at[pl.ds(base + p * _CS, _CS)],
            out_hbm.at[pl.ds(base + p * _CS, _CS)],
            csem,
        ).wait()

    @pl.when(jnp.logical_not(all_ok))
    def _fixup():
        def zinit(j, cc):
            for r in range(16):
                zrows_v[r, pl.ds(j * 16, 16)] = jnp.zeros(
                    (16,), jnp.float32
                )
            return cc

        lax.fori_loop(0, _D // 16, zinit, 0)

        def fix16(h, cc):
            lv = lab_v[pl.ds(h * 16, 16)]
            for l in range(16):
                lab = lv[l]
                bad = (lab < 0) | (lab >= _K)

                @pl.when(bad)
                def _zero_row(l=l):
                    pltpu.sync_copy(
                        zrows_v.at[pl.ds(0, 1)],
                        out_hbm.at[pl.ds(base + h * 16 + l, 1)],
                    )

            return cc

        lax.fori_loop(0, _RPW // 16, fix16, 0)


def kernel(x, y):
    xi = x.astype(jnp.int32)
    return _sc_select(xi, y)


# speedup vs baseline: 36.0628x; 36.0628x over previous
import functools

import jax
import jax.numpy as jnp
from jax import lax
from jax.experimental import pallas as pl
from jax.experimental.pallas import tpu as pltpu
from jax.experimental.pallas import tpu_sc as plsc

_K = 64
_N = 32768
_D = 1024
_NC = 2
_NS = 16
_NW = _NC * _NS
_RPW = _N // _NW
_C = 8
_NBUF = 8
_NCHUNK = _RPW // _C
_NGRP = _NCHUNK // _NBUF

_mesh = plsc.VectorSubcoreMesh(core_axis_name="c", subcore_axis_name="s")


@functools.partial(
    pl.kernel,
    out_type=jax.ShapeDtypeStruct((_N, _D), jnp.float32),
    mesh=_mesh,
    scratch_types=[
        pltpu.VMEM((_RPW,), jnp.int32),
        [pltpu.VMEM((_C, _D), jnp.float32) for _ in range(_NBUF)],
        [pltpu.SemaphoreType.DMA for _ in range(_NBUF)],
        [pltpu.SemaphoreType.DMA for _ in range(_NBUF)],
    ],
)
def _sc_select(x_hbm, y_hbm, out_hbm, lab_v, rows, gsem, ssem):
    wid = lax.axis_index("s") * _NC + lax.axis_index("c")
    base = wid * _RPW
    pltpu.sync_copy(x_hbm.at[pl.ds(base, _RPW)], lab_v)

    for b in range(_NBUF):
        pltpu.async_copy(
            y_hbm.at[pl.ds(base + b * _C, _C)], rows[b], gsem[b]
        )

    def group(go, carry):
        for b in range(_NBUF):
            g = go * _NBUF + b
            row0 = base + g * _C
            pltpu.make_async_copy(
                y_hbm.at[pl.ds(row0, _C)], rows[b], gsem[b]
            ).wait()
            pltpu.async_copy(
                rows[b], out_hbm.at[pl.ds(row0, _C)], ssem[b]
            )

            @pl.when(go < _NGRP - 1)
            def _prefetch():
                pltpu.make_async_copy(
                    rows[b], out_hbm.at[pl.ds(row0, _C)], ssem[b]
                ).wait()
                pltpu.async_copy(
                    y_hbm.at[pl.ds(row0 + _NBUF * _C, _C)],
                    rows[b],
                    gsem[b],
                )

        return carry

    lax.fori_loop(0, _NGRP, group, 0)

    def scan16(i, acc):
        lv = lab_v[pl.ds(i * 16, 16)]
        ok = jnp.where((lv >= 0) & (lv < _K), 1, 0)
        return acc & ok

    all_ok16 = lax.fori_loop(
        0, _RPW // 16, scan16, jnp.ones((16,), jnp.int32)
    )
    ok_s = all_ok16[0]
    for l in range(1, 16):
        ok_s = ok_s & all_ok16[l]
    all_ok = ok_s == 1

    for b in range(_NBUF):
        g = _NCHUNK - _NBUF + b
        pltpu.make_async_copy(
            rows[b], out_hbm.at[pl.ds(base + g * _C, _C)], ssem[b]
        ).wait()

    @pl.when(jnp.logical_not(all_ok))
    def _fixup():
        def zinit(j, cc):
            rows[0][0, pl.ds(j * 16, 16)] = jnp.zeros((16,), jnp.float32)
            return cc

        lax.fori_loop(0, _D // 16, zinit, 0)

        def fix16(h, cc):
            lv = lab_v[pl.ds(h * 16, 16)]
            for l in range(16):
                lab = lv[l]
                bad = (lab < 0) | (lab >= _K)

                @pl.when(bad)
                def _zero_row(l=l):
                    pltpu.sync_copy(
                        rows[0].at[pl.ds(0, 1)],
                        out_hbm.at[pl.ds(base + h * 16 + l, 1)],
                    )

            return cc

        lax.fori_loop(0, _RPW // 16, fix16, 0)


def kernel(x, y):
    xi = x.astype(jnp.int32)
    return _sc_select(xi, y)
